# Initial kernel scaffold; baseline (speedup 1.0000x reference)
import jax
import jax.numpy as jnp
from jax.experimental import pallas as pl


def kernel(q, k, memory):
    raise NotImplementedError("write your pallas kernel here")


# speedup vs baseline: 5.3456x; 5.3456x over previous
import functools

import jax
import jax.numpy as jnp
from jax.experimental import pallas as pl
from jax.experimental.pallas import tpu as pltpu

T = 0.05
KNN = 50
W_POS = 0.2

RB = 32
CQ = 2048
N_BISECT = 32


def _loss_kernel(q_ref, k_ref, mem_ref, out_ref, logits_ref, *, n_chunks, queue):
    j = pl.program_id(1)
    qb = q_ref[...]
    mb = mem_ref[...]
    lg = jax.lax.dot_general(
        qb, mb, (((1,), (1,)), ((), ())),
        preferred_element_type=jnp.float32) * (1.0 / T)
    col = j * CQ + jax.lax.broadcasted_iota(jnp.int32, (RB, CQ), 1)
    lg = jnp.where(col < queue, lg, -1e30)
    logits_ref[j] = lg

    @pl.when(j == n_chunks - 1)
    def _finish():
        x = logits_ref[...]
        lpos = jnp.sum(q_ref[...] * k_ref[...], axis=1) * (1.0 / T)

        m = jnp.maximum(jnp.max(x, axis=(0, 2)), lpos)
        se = (jnp.sum(jnp.exp(x - m[None, :, None]), axis=(0, 2))
              + jnp.exp(lpos - m))
        lse = m + jnp.log(se)

        xr = jnp.where(x > -5e29, x, 1e30)
        lo = jnp.minimum(jnp.min(xr, axis=(0, 2)), lpos) - 1.0
        hi = m

        def body(_, carry):
            lo, hi = carry
            mid = 0.5 * (lo + hi)
            c = (jnp.sum((x > mid[None, :, None]).astype(jnp.float32),
                         axis=(0, 2))
                 + (lpos > mid).astype(jnp.float32))
            ge = c >= KNN
            return jnp.where(ge, mid, lo), jnp.where(ge, hi, mid)

        lo, hi = jax.lax.fori_loop(0, N_BISECT, body, (lo, hi))

        above = x > hi[None, :, None]
        chi = (jnp.sum(above.astype(jnp.float32), axis=(0, 2))
               + (lpos > hi).astype(jnp.float32))
        s_top = (jnp.sum(jnp.where(above, x, 0.0), axis=(0, 2))
                 + jnp.where(lpos > hi, lpos, 0.0)
                 + (KNN - chi) * hi)
        ind = (lpos > hi).astype(jnp.float32)

        w = (1.0 - W_POS) / KNN
        tsum = 1.0 - w * ind
        tdot = W_POS * lpos + w * (s_top - ind * lpos)
        out_ref[...] = (tsum * lse - tdot)[:, None]


def kernel(q, k, memory):
    b, d = q.shape
    queue = memory.shape[0]
    n_chunks = -(-queue // CQ)
    qpad = n_chunks * CQ
    mem_p = jnp.pad(memory, ((0, qpad - queue), (0, 0)))

    losses = pl.pallas_call(
        functools.partial(_loss_kernel, n_chunks=n_chunks, queue=queue),
        grid=(b // RB, n_chunks),
        in_specs=[
            pl.BlockSpec((RB, d), lambda i, j: (i, 0)),
            pl.BlockSpec((RB, d), lambda i, j: (i, 0)),
            pl.BlockSpec((CQ, d), lambda i, j: (j, 0)),
        ],
        out_specs=pl.BlockSpec((RB, 1), lambda i, j: (i, 0)),
        out_shape=jax.ShapeDtypeStruct((b, 1), jnp.float32),
        scratch_shapes=[pltpu.VMEM((n_chunks, RB, CQ), jnp.float32)],
        compiler_params=pltpu.CompilerParams(
            dimension_semantics=("parallel", "arbitrary")),
    )(q, k, mem_p)
    return jnp.mean(losses)


# speedup vs baseline: 7.6624x; 1.4334x over previous
import functools

import jax
import jax.numpy as jnp
from jax.experimental import pallas as pl
from jax.experimental.pallas import tpu as pltpu

T = 0.05
KNN = 50
W_POS = 0.2

RB = 64
CQ = 2048
N_BISECT = 32


def _loss_kernel(q_ref, k_ref, mem_ref, out_ref, logits_ref, *, n_chunks, queue):
    j = pl.program_id(1)
    qb = q_ref[...]
    mb = mem_ref[...]
    lg = jax.lax.dot_general(
        qb, mb, (((1,), (1,)), ((), ())),
        preferred_element_type=jnp.float32) * (1.0 / T)
    col = j * CQ + jax.lax.broadcasted_iota(jnp.int32, (RB, CQ), 1)
    lg = jnp.where(col < queue, lg, -1e30)
    logits_ref[j] = lg

    @pl.when(j == n_chunks - 1)
    def _finish():
        x = logits_ref[...]
        lpos = jnp.sum(q_ref[...] * k_ref[...], axis=1) * (1.0 / T)

        m = jnp.maximum(jnp.max(x, axis=(0, 2)), lpos)
        se = (jnp.sum(jnp.exp(x - m[None, :, None]), axis=(0, 2))
              + jnp.exp(lpos - m))
        lse = m + jnp.log(se)

        xr = jnp.where(x > -5e29, x, 1e30)
        lo = jnp.minimum(jnp.min(xr, axis=(0, 2)), lpos) - 1.0
        hi = m

        def body(_, carry):
            lo, hi = carry
            mid = 0.5 * (lo + hi)
            c = (jnp.sum((x > mid[None, :, None]).astype(jnp.float32),
                         axis=(0, 2))
                 + (lpos > mid).astype(jnp.float32))
            ge = c >= KNN
            return jnp.where(ge, mid, lo), jnp.where(ge, hi, mid)

        lo, hi = jax.lax.fori_loop(0, N_BISECT, body, (lo, hi))

        above = x > hi[None, :, None]
        chi = (jnp.sum(above.astype(jnp.float32), axis=(0, 2))
               + (lpos > hi).astype(jnp.float32))
        s_top = (jnp.sum(jnp.where(above, x, 0.0), axis=(0, 2))
                 + jnp.where(lpos > hi, lpos, 0.0)
                 + (KNN - chi) * hi)
        ind = (lpos > hi).astype(jnp.float32)

        w = (1.0 - W_POS) / KNN
        tsum = 1.0 - w * ind
        tdot = W_POS * lpos + w * (s_top - ind * lpos)
        out_ref[...] = (tsum * lse - tdot)[:, None]


def kernel(q, k, memory):
    b, d = q.shape
    queue = memory.shape[0]
    n_chunks = -(-queue // CQ)
    qpad = n_chunks * CQ
    mem_p = jnp.pad(memory, ((0, qpad - queue), (0, 0)))

    losses = pl.pallas_call(
        functools.partial(_loss_kernel, n_chunks=n_chunks, queue=queue),
        grid=(b // RB, n_chunks),
        in_specs=[
            pl.BlockSpec((RB, d), lambda i, j: (i, 0)),
            pl.BlockSpec((RB, d), lambda i, j: (i, 0)),
            pl.BlockSpec((CQ, d), lambda i, j: (j, 0)),
        ],
        out_specs=pl.BlockSpec((RB, 1), lambda i, j: (i, 0)),
        out_shape=jax.ShapeDtypeStruct((b, 1), jnp.float32),
        scratch_shapes=[pltpu.VMEM((n_chunks, RB, CQ), jnp.float32)],
        compiler_params=pltpu.CompilerParams(
            dimension_semantics=("parallel", "arbitrary")),
    )(q, k, mem_p)
    return jnp.mean(losses)


# speedup vs baseline: 11.0954x; 1.4480x over previous
import functools

import jax
import jax.numpy as jnp
from jax.experimental import pallas as pl
from jax.experimental.pallas import tpu as pltpu

T = 0.05
KNN = 50
W_POS = 0.2

RB = 64
CQ = 2048
N_BISECT_CM = 26
N_BISECT = 16


def _loss_kernel(q_ref, k_ref, mem_ref, out_ref, logits_ref, colmax_ref,
                 *, n_chunks, queue):
    j = pl.program_id(1)
    qb = q_ref[...]
    mb = mem_ref[...]
    lg = jax.lax.dot_general(
        qb, mb, (((1,), (1,)), ((), ())),
        preferred_element_type=jnp.float32) * (1.0 / T)
    col = j * CQ + jax.lax.broadcasted_iota(jnp.int32, (RB, CQ), 1)
    lg = jnp.where(col < queue, lg, -1e30)
    logits_ref[j] = lg

    @pl.when(j == 0)
    def _init():
        colmax_ref[...] = lg

    @pl.when(j > 0)
    def _acc():
        colmax_ref[...] = jnp.maximum(colmax_ref[...], lg)

    @pl.when(j == n_chunks - 1)
    def _finish():
        x = logits_ref[...]
        cm = colmax_ref[...]
        lpos = jnp.sum(q_ref[...] * k_ref[...], axis=1) * (1.0 / T)

        m = jnp.maximum(jnp.max(cm, axis=1), lpos)
        se = (jnp.sum(jnp.exp(x - m[None, :, None]), axis=(0, 2))
              + jnp.exp(lpos - m))
        lse = m + jnp.log(se)

        cmin = jnp.min(jnp.where(cm > -5e29, cm, 1e30), axis=1) - 1.0
        hi0 = jnp.max(cm, axis=1)

        def body_cm(_, carry):
            lo, hi = carry
            mid = 0.5 * (lo + hi)
            c = jnp.sum((cm > mid[:, None]).astype(jnp.float32), axis=1)
            ge = c >= KNN
            return jnp.where(ge, mid, lo), jnp.where(ge, hi, mid)

        lo, _ = jax.lax.fori_loop(0, N_BISECT_CM, body_cm, (cmin, hi0))
        hi = m

        def body(_, carry):
            lo, hi = carry
            mid = 0.5 * (lo + hi)
            c = (jnp.sum((x > mid[None, :, None]).astype(jnp.float32),
                         axis=(0, 2))
                 + (lpos > mid).astype(jnp.float32))
            ge = c >= KNN
            return jnp.where(ge, mid, lo), jnp.where(ge, hi, mid)

        lo, hi = jax.lax.fori_loop(0, N_BISECT, body, (lo, hi))

        above = x > hi[None, :, None]
        chi = (jnp.sum(above.astype(jnp.float32), axis=(0, 2))
               + (lpos > hi).astype(jnp.float32))
        s_top = (jnp.sum(jnp.where(above, x, 0.0), axis=(0, 2))
                 + jnp.where(lpos > hi, lpos, 0.0)
                 + (KNN - chi) * hi)
        ind = (lpos > hi).astype(jnp.float32)

        w = (1.0 - W_POS) / KNN
        tsum = 1.0 - w * ind
        tdot = W_POS * lpos + w * (s_top - ind * lpos)
        out_ref[...] = (tsum * lse - tdot)[:, None]


def kernel(q, k, memory):
    b, d = q.shape
    queue = memory.shape[0]
    n_chunks = -(-queue // CQ)
    qpad = n_chunks * CQ
    mem_p = jnp.pad(memory, ((0, qpad - queue), (0, 0)))

    losses = pl.pallas_call(
        functools.partial(_loss_kernel, n_chunks=n_chunks, queue=queue),
        grid=(b // RB, n_chunks),
        in_specs=[
            pl.BlockSpec((RB, d), lambda i, j: (i, 0)),
            pl.BlockSpec((RB, d), lambda i, j: (i, 0)),
            pl.BlockSpec((CQ, d), lambda i, j: (j, 0)),
        ],
        out_specs=pl.BlockSpec((RB, 1), lambda i, j: (i, 0)),
        out_shape=jax.ShapeDtypeStruct((b, 1), jnp.float32),
        scratch_shapes=[pltpu.VMEM((n_chunks, RB, CQ), jnp.float32),
                        pltpu.VMEM((RB, CQ), jnp.float32)],
        compiler_params=pltpu.CompilerParams(
            dimension_semantics=("parallel", "arbitrary")),
    )(q, k, mem_p)
    return jnp.mean(losses)


# speedup vs baseline: 12.6217x; 1.1376x over previous
import functools

import jax
import jax.numpy as jnp
from jax.experimental import pallas as pl
from jax.experimental.pallas import tpu as pltpu

T = 0.05
KNN = 50
W_POS = 0.2

RB = 64
CQ = 2048
N_BISECT_CM = 26
N_BISECT = 10


def _loss_kernel(q_ref, k_ref, mem_ref, out_ref, logits_ref, colmax_ref,
                 mrun_ref, serun_ref, *, n_chunks, queue):
    j = pl.program_id(1)
    qb = q_ref[...]
    mb = mem_ref[...]
    lg = jax.lax.dot_general(
        qb, mb, (((1,), (1,)), ((), ())),
        preferred_element_type=jnp.float32) * (1.0 / T)
    col = j * CQ + jax.lax.broadcasted_iota(jnp.int32, (RB, CQ), 1)
    lg = jnp.where(col < queue, lg, -1e30)
    logits_ref[j] = lg

    cmx = jnp.max(lg, axis=1, keepdims=True)

    @pl.when(j == 0)
    def _init():
        colmax_ref[...] = lg
        mrun_ref[...] = cmx
        serun_ref[...] = jnp.sum(jnp.exp(lg - cmx), axis=1, keepdims=True)

    @pl.when(j > 0)
    def _acc():
        colmax_ref[...] = jnp.maximum(colmax_ref[...], lg)
        mold = mrun_ref[...]
        mnew = jnp.maximum(mold, cmx)
        serun_ref[...] = (serun_ref[...] * jnp.exp(mold - mnew)
                          + jnp.sum(jnp.exp(lg - mnew), axis=1, keepdims=True))
        mrun_ref[...] = mnew

    @pl.when(j == n_chunks - 1)
    def _finish():
        x = logits_ref[...]
        cm = colmax_ref[...]
        lpos = jnp.sum(q_ref[...] * k_ref[...], axis=1) * (1.0 / T)

        mr = mrun_ref[...][:, 0]
        m = jnp.maximum(mr, lpos)
        se = (serun_ref[...][:, 0] * jnp.exp(mr - m) + jnp.exp(lpos - m))
        lse = m + jnp.log(se)

        cmin = jnp.min(jnp.where(cm > -5e29, cm, 1e30), axis=1) - 1.0
        hi0 = jnp.max(cm, axis=1)

        def body_cm(_, carry):
            lo, hi = carry
            mid = 0.5 * (lo + hi)
            c = jnp.sum((cm > mid[:, None]).astype(jnp.float32), axis=1)
            ge = c >= KNN
            return jnp.where(ge, mid, lo), jnp.where(ge, hi, mid)

        lo, _ = jax.lax.fori_loop(0, N_BISECT_CM, body_cm, (cmin, hi0))
        hi = m

        def body(_, carry):
            lo, hi = carry
            mid = 0.5 * (lo + hi)
            c = (jnp.sum((x > mid[None, :, None]).astype(jnp.float32),
                         axis=(0, 2))
                 + (lpos > mid).astype(jnp.float32))
            ge = c >= KNN
            return jnp.where(ge, mid, lo), jnp.where(ge, hi, mid)

        lo, hi = jax.lax.fori_loop(0, N_BISECT, body, (lo, hi))

        above = x > hi[None, :, None]
        chi = (jnp.sum(above.astype(jnp.float32), axis=(0, 2))
               + (lpos > hi).astype(jnp.float32))
        s_top = (jnp.sum(jnp.where(above, x, 0.0), axis=(0, 2))
                 + jnp.where(lpos > hi, lpos, 0.0)
                 + (KNN - chi) * hi)
        ind = (lpos > hi).astype(jnp.float32)

        w = (1.0 - W_POS) / KNN
        tsum = 1.0 - w * ind
        tdot = W_POS * lpos + w * (s_top - ind * lpos)
        out_ref[...] = (tsum * lse - tdot)[:, None]


def kernel(q, k, memory):
    b, d = q.shape
    queue = memory.shape[0]
    n_chunks = -(-queue // CQ)
    qpad = n_chunks * CQ
    mem_p = jnp.pad(memory, ((0, qpad - queue), (0, 0)))

    losses = pl.pallas_call(
        functools.partial(_loss_kernel, n_chunks=n_chunks, queue=queue),
        grid=(b // RB, n_chunks),
        in_specs=[
            pl.BlockSpec((RB, d), lambda i, j: (i, 0)),
            pl.BlockSpec((RB, d), lambda i, j: (i, 0)),
            pl.BlockSpec((CQ, d), lambda i, j: (j, 0)),
        ],
        out_specs=pl.BlockSpec((RB, 1), lambda i, j: (i, 0)),
        out_shape=jax.ShapeDtypeStruct((b, 1), jnp.float32),
        scratch_shapes=[pltpu.VMEM((n_chunks, RB, CQ), jnp.float32),
                        pltpu.VMEM((RB, CQ), jnp.float32),
                        pltpu.VMEM((RB, 1), jnp.float32),
                        pltpu.VMEM((RB, 1), jnp.float32)],
        compiler_params=pltpu.CompilerParams(
            dimension_semantics=("parallel", "arbitrary")),
    )(q, k, mem_p)
    return jnp.mean(losses)
